# Initial kernel scaffold; baseline (speedup 1.0000x reference)
import jax
import jax.numpy as jnp
from jax.experimental import pallas as pl


def kernel(x, edge_index, edge_attr, W1, b1, W2, b2, Wroot, broot, gat1_W, gat1_asrc, gat1_adst, gat1_b, gat2_W, gat2_asrc, gat2_adst, gat2_b):
    raise NotImplementedError("write your pallas kernel here")


# speedup vs baseline: 13.3266x; 13.3266x over previous
import functools

import jax
import jax.numpy as jnp
from jax import lax
from jax.experimental import pallas as pl
from jax.experimental.pallas import tpu as pltpu
from jax.experimental.pallas import tpu_sc as plsc

N = 10000
E = 80000
F_IN = 16
HID = 128
H1, C1 = 8, 16
NCLS = 16

NP = 10048
JUNK = N
NWORK = 32
G = 96
EP = 98304
GP = EP // (NWORK * G)
BE = 512
W144 = 144

def _mesh():
    return plsc.VectorSubcoreMesh(core_axis_name="c", subcore_axis_name="s")


def _sc_gather_body(tab_hbm, idx2d_hbm, out_hbm, idx_v, rows_v, sem):
    c = lax.axis_index("c")
    s = lax.axis_index("s")
    wid = s * 2 + c
    ngroups = idx2d_hbm.shape[0] // NWORK
    pltpu.sync_copy(idx2d_hbm.at[pl.ds(wid * ngroups, ngroups)], idx_v)

    def body(g, _):
        pltpu.async_copy(tab_hbm.at[idx_v.at[g]], rows_v, sem).wait()
        pltpu.sync_copy(rows_v, out_hbm.at[pl.ds(wid * ngroups * G + g * G, G)])
        return _

    lax.fori_loop(0, ngroups, body, 0, unroll=False)


def _sc_scatter_body(rows_hbm, idx2d_hbm, z_hbm, out_hbm, idx_v, rows_v, acc_sh, sem):
    c = lax.axis_index("c")
    s = lax.axis_index("s")
    wid = s * 2 + c
    ngroups = idx2d_hbm.shape[0] // NWORK
    rpt = NP // 16
    pltpu.sync_copy(z_hbm.at[pl.ds(s * rpt, rpt)], acc_sh.at[pl.ds(s * rpt, rpt)])
    pltpu.sync_copy(idx2d_hbm.at[pl.ds(wid * ngroups, ngroups)], idx_v)
    plsc.subcore_barrier()

    def body(g, _):
        pltpu.async_copy(rows_hbm.at[pl.ds(wid * ngroups * G + g * G, G)], rows_v, sem).wait()
        pltpu.sync_copy(rows_v, acc_sh.at[idx_v.at[g]], add=True)
        return _

    lax.fori_loop(0, ngroups, body, 0, unroll=False)
    plsc.subcore_barrier()
    pltpu.sync_copy(acc_sh.at[pl.ds(s * rpt, rpt)],
                    out_hbm.at[pl.ds(c * NP + s * rpt, rpt)])


def _sc_gat1_body(hw_hbm, ta_hbm, tb_hbm, isrc_hbm, idst_hbm, z_hbm, out_hbm,
                  isrc_v, idst_v, g1_v, g2_v, hw_v, rows_v, acc_sh, sem):
    c = lax.axis_index("c")
    s = lax.axis_index("s")
    wid = s * 2 + c
    rpt = NP // 16
    pltpu.sync_copy(z_hbm.at[pl.ds(s * rpt, rpt)], acc_sh.at[pl.ds(s * rpt, rpt)])
    pltpu.sync_copy(isrc_hbm.at[pl.ds(wid * GP, GP)], isrc_v)
    pltpu.sync_copy(idst_hbm.at[pl.ds(wid * GP, GP)], idst_v)
    plsc.subcore_barrier()

    def group(g, _):
        pltpu.async_copy(ta_hbm.at[isrc_v.at[g]], g1_v, sem).wait()
        pltpu.async_copy(tb_hbm.at[idst_v.at[g]], g2_v, sem).wait()
        pltpu.async_copy(hw_hbm.at[isrc_v.at[g]], hw_v, sem).wait()

        def edge(i, _):
            asum = g1_v[i, :] + g2_v[i, :]
            al = jnp.where(asum >= 0.0, asum, 0.2 * asum)
            aexp = jnp.exp(al)
            rows_v[i, pl.ds(128, 16)] = aexp
            for h in range(H1):
                hidx = jnp.full((16, 1), h, jnp.int32)
                wv = lax.gather(
                    aexp, hidx,
                    lax.GatherDimensionNumbers(
                        offset_dims=(), collapsed_slice_dims=(0,),
                        start_index_map=(0,)),
                    slice_sizes=(1,),
                    mode=lax.GatherScatterMode.PROMISE_IN_BOUNDS)
                rows_v[i, pl.ds(h * 16, 16)] = hw_v[i, pl.ds(h * 16, 16)] * wv
            return _

        lax.fori_loop(0, G, edge, 0, unroll=False)
        pltpu.sync_copy(rows_v, acc_sh.at[idst_v.at[g]], add=True)
        return _

    lax.fori_loop(0, GP, group, 0, unroll=False)
    plsc.subcore_barrier()
    pltpu.sync_copy(acc_sh.at[pl.ds(s * rpt, rpt)],
                    out_hbm.at[pl.ds(c * NP + s * rpt, rpt)])


def _sc_gat2_body(hw_hbm, ta_hbm, tb_hbm, isrc_hbm, idst_hbm, z_hbm, out_hbm,
                  isrc_v, idst_v, g1_v, g2_v, hw_v, rows_v, acc_sh, sem):
    c = lax.axis_index("c")
    s = lax.axis_index("s")
    wid = s * 2 + c
    rpt = NP // 16
    pltpu.sync_copy(z_hbm.at[pl.ds(s * rpt, rpt)], acc_sh.at[pl.ds(s * rpt, rpt)])
    pltpu.sync_copy(isrc_hbm.at[pl.ds(wid * GP, GP)], isrc_v)
    pltpu.sync_copy(idst_hbm.at[pl.ds(wid * GP, GP)], idst_v)
    plsc.subcore_barrier()

    def group(g, _):
        pltpu.async_copy(ta_hbm.at[isrc_v.at[g]], g1_v, sem).wait()
        pltpu.async_copy(tb_hbm.at[idst_v.at[g]], g2_v, sem).wait()
        pltpu.async_copy(hw_hbm.at[isrc_v.at[g]], hw_v, sem).wait()

        def edge(i, _):
            asum = g1_v[i, :] + g2_v[i, :]
            al = jnp.where(asum >= 0.0, asum, 0.2 * asum)
            aexp = jnp.exp(al)
            rows_v[i, pl.ds(0, 16)] = hw_v[i, :] * aexp
            rows_v[i, pl.ds(16, 16)] = aexp
            return _

        lax.fori_loop(0, G, edge, 0, unroll=False)
        pltpu.sync_copy(rows_v, acc_sh.at[idst_v.at[g]], add=True)
        return _

    lax.fori_loop(0, GP, group, 0, unroll=False)
    plsc.subcore_barrier()
    pltpu.sync_copy(acc_sh.at[pl.ds(s * rpt, rpt)],
                    out_hbm.at[pl.ds(c * NP + s * rpt, rpt)])


def _tc_msg_body(ea_ref, xs_ref, w1_ref, b1_ref, w2r_ref, b2r_ref, out_ref):
    h1 = jnp.maximum(
        jnp.dot(ea_ref[...], w1_ref[...], preferred_element_type=jnp.float32)
        + b1_ref[...], 0.0)
    xs = xs_ref[...]
    a = jnp.concatenate([xs[:, f:f + 1] * h1 for f in range(F_IN)], axis=1)
    msg = jnp.dot(a, w2r_ref[...], preferred_element_type=jnp.float32)
    msg = msg + jnp.dot(xs, b2r_ref[...], preferred_element_type=jnp.float32)
    ones = jnp.ones((msg.shape[0], 1), jnp.float32)
    zer = jnp.zeros((msg.shape[0], W144 - HID - 1), jnp.float32)
    out_ref[...] = jnp.concatenate([msg, ones, zer], axis=1)


def _tc_h_gat1_body(acc_ref, x_ref, wroot_ref, broot_ref, w_ref, as_ref, ad_ref,
                    hw_ref, ta_ref, tb_ref):
    acc = acc_ref[0] + acc_ref[1]
    deg = jnp.maximum(acc[:, HID:HID + 1], 1.0)
    h = jnp.maximum(
        acc[:, :HID] / deg
        + jnp.dot(x_ref[...], wroot_ref[...], preferred_element_type=jnp.float32)
        + broot_ref[...], 0.0)
    hw = jnp.dot(h, w_ref[...], preferred_element_type=jnp.float32)
    hw_ref[...] = hw
    ta_ref[...] = jnp.dot(hw, as_ref[...], preferred_element_type=jnp.float32)
    tb_ref[...] = jnp.dot(hw, ad_ref[...], preferred_element_type=jnp.float32)


def _tc_gat2_prep_body(acc_ref, b_ref, w_ref, as_ref, ad_ref,
                       hw_ref, ta_ref, tb_ref):
    acc = acc_ref[0] + acc_ref[1]
    s = acc[:, :HID]
    d = acc[:, HID:HID + H1]
    dfull = jnp.repeat(d, C1, axis=1)
    g1 = s / (dfull + 1e-16) + b_ref[...]
    h2 = jnp.where(g1 > 0.0, g1, jnp.exp(jnp.minimum(g1, 0.0)) - 1.0)
    hw = jnp.dot(h2, w_ref[...], preferred_element_type=jnp.float32)
    hw_ref[...] = hw
    ta_ref[...] = jnp.dot(hw, as_ref[...], preferred_element_type=jnp.float32)
    tb_ref[...] = jnp.dot(hw, ad_ref[...], preferred_element_type=jnp.float32)


def _tc_final_body(acc_ref, b_ref, out_ref):
    acc = acc_ref[0] + acc_ref[1]
    s = acc[:, :NCLS]
    d = acc[:, NCLS:NCLS + 1]
    o = s / (d + 1e-16) + b_ref[...]
    m = jnp.max(o, axis=1, keepdims=True)
    z = o - m
    lse = jnp.log(jnp.sum(jnp.exp(z), axis=1, keepdims=True))
    out_ref[...] = z - lse


def _sc_gather(tab, idx2d):
    k = functools.partial(
        pl.kernel,
        out_type=jax.ShapeDtypeStruct((EP, F_IN), jnp.float32),
        scratch_types=[
            pltpu.VMEM((GP, G), jnp.int32),
            pltpu.VMEM((G, F_IN), jnp.float32),
            pltpu.SemaphoreType.DMA,
        ],
        mesh=_mesh(),
        compiler_params=pltpu.CompilerParams(use_tc_tiling_on_sc=False),
    )(_sc_gather_body)
    return k(tab, idx2d)


def _sc_scatter(rows, idx2d, z):
    k = functools.partial(
        pl.kernel,
        out_type=jax.ShapeDtypeStruct((2 * NP, W144), jnp.float32),
        scratch_types=[
            pltpu.VMEM((GP, G), jnp.int32),
            pltpu.VMEM((G, W144), jnp.float32),
            pltpu.VMEM_SHARED((NP, W144), jnp.float32),
            pltpu.SemaphoreType.DMA,
        ],
        mesh=_mesh(),
        compiler_params=pltpu.CompilerParams(use_tc_tiling_on_sc=False),
    )(_sc_scatter_body)
    return k(rows, idx2d, z)


def _sc_gat(body, hw, ta, tb, isrc, idst, z, width):
    k = functools.partial(
        pl.kernel,
        out_type=jax.ShapeDtypeStruct((2 * NP, width), jnp.float32),
        scratch_types=[
            pltpu.VMEM((GP, G), jnp.int32),
            pltpu.VMEM((GP, G), jnp.int32),
            pltpu.VMEM((G, 16), jnp.float32),
            pltpu.VMEM((G, 16), jnp.float32),
            pltpu.VMEM((G, hw.shape[1]), jnp.float32),
            pltpu.VMEM((G, width), jnp.float32),
            pltpu.VMEM_SHARED((NP, width), jnp.float32),
            pltpu.SemaphoreType.DMA,
        ],
        mesh=_mesh(),
        compiler_params=pltpu.CompilerParams(use_tc_tiling_on_sc=False),
    )(body)
    return k(hw, ta, tb, isrc, idst, z)


def kernel(x, edge_index, edge_attr, W1, b1, W2, b2, Wroot, broot,
           gat1_W, gat1_asrc, gat1_adst, gat1_b,
           gat2_W, gat2_asrc, gat2_adst, gat2_b):
    f32 = jnp.float32
    src = edge_index[0]
    dst = edge_index[1]

    pad1 = EP - E
    src_p = jnp.concatenate([src, jnp.zeros((pad1,), src.dtype)])
    dst_p = jnp.concatenate([dst, jnp.full((pad1,), JUNK, dst.dtype)])
    ea_p = jnp.concatenate([edge_attr, jnp.zeros((pad1, F_IN), f32)])
    xp = jnp.concatenate([x, jnp.zeros((NP - N, F_IN), f32)])

    pad2 = EP - (E + N)
    loop = jnp.arange(N, dtype=src.dtype)
    src2 = jnp.concatenate([src, loop, jnp.zeros((pad2,), src.dtype)])
    dst2 = jnp.concatenate([dst, loop, jnp.full((pad2,), JUNK, dst.dtype)])

    W2r = W2.reshape(HID, F_IN, HID).transpose(1, 0, 2).reshape(F_IN * HID, HID)
    B2 = b2.reshape(F_IN, HID)
    b1r = b1.reshape(1, HID)
    brootr = broot.reshape(1, HID)
    g1br = gat1_b.reshape(1, H1 * C1)
    g2br = gat2_b.reshape(1, NCLS)

    hh = jnp.arange(H1)
    As3 = jnp.zeros((H1, C1, H1), f32).at[hh, :, hh].set(gat1_asrc)
    Ad3 = jnp.zeros((H1, C1, H1), f32).at[hh, :, hh].set(gat1_adst)
    ASrep = jnp.concatenate([As3.reshape(HID, H1)] * 2, axis=1)
    ADrep = jnp.concatenate([Ad3.reshape(HID, H1)] * 2, axis=1)
    As2rep = jnp.tile(gat2_asrc.reshape(NCLS, 1), (1, 16))
    Ad2rep = jnp.tile(gat2_adst.reshape(NCLS, 1), (1, 16))

    z144 = jnp.zeros((NP, W144), f32)
    z32 = jnp.zeros((NP, 32), f32)

    src_p2d = src_p.reshape(EP // G, G)
    dst_p2d = dst_p.reshape(EP // G, G)
    src2_2d = src2.reshape(EP // G, G)
    dst2_2d = dst2.reshape(EP // G, G)

    xs = _sc_gather(xp, src_p2d)

    nb = -(-E // BE)
    msg = pl.pallas_call(
        _tc_msg_body,
        grid=(nb,),
        in_specs=[
            pl.BlockSpec((BE, F_IN), lambda i: (i, 0)),
            pl.BlockSpec((BE, F_IN), lambda i: (i, 0)),
            pl.BlockSpec((F_IN, HID), lambda i: (0, 0)),
            pl.BlockSpec((1, HID), lambda i: (0, 0)),
            pl.BlockSpec((F_IN * HID, HID), lambda i: (0, 0)),
            pl.BlockSpec((F_IN, HID), lambda i: (0, 0)),
        ],
        out_specs=pl.BlockSpec((BE, W144), lambda i: (i, 0)),
        out_shape=jax.ShapeDtypeStruct((EP, W144), f32),
    )(ea_p, xs, W1, b1r, W2r, B2)

    acc1 = _sc_scatter(msg, dst_p2d, z144).reshape(2, NP, W144)

    nb3 = NP // 2512
    hw1, ta1, tb1 = pl.pallas_call(
        _tc_h_gat1_body,
        grid=(nb3,),
        in_specs=[
            pl.BlockSpec((2, 2512, W144), lambda i: (0, i, 0)),
            pl.BlockSpec((2512, F_IN), lambda i: (i, 0)),
            pl.BlockSpec((F_IN, HID), lambda i: (0, 0)),
            pl.BlockSpec((1, HID), lambda i: (0, 0)),
            pl.BlockSpec((HID, HID), lambda i: (0, 0)),
            pl.BlockSpec((HID, 16), lambda i: (0, 0)),
            pl.BlockSpec((HID, 16), lambda i: (0, 0)),
        ],
        out_specs=[
            pl.BlockSpec((2512, HID), lambda i: (i, 0)),
            pl.BlockSpec((2512, 16), lambda i: (i, 0)),
            pl.BlockSpec((2512, 16), lambda i: (i, 0)),
        ],
        out_shape=[
            jax.ShapeDtypeStruct((NP, HID), f32),
            jax.ShapeDtypeStruct((NP, 16), f32),
            jax.ShapeDtypeStruct((NP, 16), f32),
        ],
    )(acc1, xp, Wroot, brootr, gat1_W, ASrep, ADrep)

    acc2 = _sc_gat(_sc_gat1_body, hw1, ta1, tb1, src2_2d, dst2_2d, z144,
                   W144).reshape(2, NP, W144)

    hw2, ta2, tb2 = pl.pallas_call(
        _tc_gat2_prep_body,
        grid=(nb3,),
        in_specs=[
            pl.BlockSpec((2, 2512, W144), lambda i: (0, i, 0)),
            pl.BlockSpec((1, HID), lambda i: (0, 0)),
            pl.BlockSpec((HID, NCLS), lambda i: (0, 0)),
            pl.BlockSpec((NCLS, 16), lambda i: (0, 0)),
            pl.BlockSpec((NCLS, 16), lambda i: (0, 0)),
        ],
        out_specs=[
            pl.BlockSpec((2512, NCLS), lambda i: (i, 0)),
            pl.BlockSpec((2512, 16), lambda i: (i, 0)),
            pl.BlockSpec((2512, 16), lambda i: (i, 0)),
        ],
        out_shape=[
            jax.ShapeDtypeStruct((NP, NCLS), f32),
            jax.ShapeDtypeStruct((NP, 16), f32),
            jax.ShapeDtypeStruct((NP, 16), f32),
        ],
    )(acc2, g1br, gat2_W, As2rep, Ad2rep)

    acc3 = _sc_gat(_sc_gat2_body, hw2, ta2, tb2, src2_2d, dst2_2d, z32,
                   32).reshape(2, NP, 32)

    out = pl.pallas_call(
        _tc_final_body,
        grid=(nb3,),
        in_specs=[
            pl.BlockSpec((2, 2512, 32), lambda i: (0, i, 0)),
            pl.BlockSpec((1, NCLS), lambda i: (0, 0)),
        ],
        out_specs=pl.BlockSpec((2512, NCLS), lambda i: (i, 0)),
        out_shape=jax.ShapeDtypeStruct((NP, NCLS), f32),
    )(acc3, g2br)

    return out[:N]


# speedup vs baseline: 14.4813x; 1.0866x over previous
import functools

import jax
import jax.numpy as jnp
from jax import lax
from jax.experimental import pallas as pl
from jax.experimental.pallas import tpu as pltpu
from jax.experimental.pallas import tpu_sc as plsc

N = 10000
E = 80000
F_IN = 16
HID = 128
H1, C1 = 8, 16
NCLS = 16

NP = 10048
JUNK = N
NWORK = 32
G = 96
EP = 98304
GP = EP // (NWORK * G)
BE = 512
W144 = 144

def _mesh():
    return plsc.VectorSubcoreMesh(core_axis_name="c", subcore_axis_name="s")


def _sc_gather_body(tab_hbm, idx2d_hbm, out_hbm, idx_v, rows_v, sem):
    c = lax.axis_index("c")
    s = lax.axis_index("s")
    wid = s * 2 + c
    ngroups = idx2d_hbm.shape[0] // NWORK
    pltpu.sync_copy(idx2d_hbm.at[pl.ds(wid * ngroups, ngroups)], idx_v)
    pltpu.async_copy(tab_hbm.at[idx_v.at[0]], rows_v.at[0], sem).wait()

    def body(g, _):
        nxt = pltpu.async_copy(tab_hbm.at[idx_v.at[g + 1]], rows_v.at[(g + 1) % 2], sem)
        pltpu.sync_copy(rows_v.at[g % 2],
                        out_hbm.at[pl.ds(wid * ngroups * G + g * G, G)])
        nxt.wait()
        return _

    lax.fori_loop(0, ngroups - 1, body, 0, unroll=False)
    g = ngroups - 1
    pltpu.sync_copy(rows_v.at[g % 2],
                    out_hbm.at[pl.ds(wid * ngroups * G + g * G, G)])


def _sc_scatter_body(rows_hbm, idx2d_hbm, z_hbm, out_hbm, idx_v, rows_v, acc_sh, sem):
    c = lax.axis_index("c")
    s = lax.axis_index("s")
    wid = s * 2 + c
    ngroups = idx2d_hbm.shape[0] // NWORK
    rpt = NP // 16
    pltpu.sync_copy(z_hbm.at[pl.ds(s * rpt, rpt)], acc_sh.at[pl.ds(s * rpt, rpt)])
    pltpu.sync_copy(idx2d_hbm.at[pl.ds(wid * ngroups, ngroups)], idx_v)
    plsc.subcore_barrier()

    def body(g, _):
        pltpu.async_copy(rows_hbm.at[pl.ds(wid * ngroups * G + g * G, G)], rows_v, sem).wait()
        pltpu.sync_copy(rows_v, acc_sh.at[idx_v.at[g]], add=True)
        return _

    lax.fori_loop(0, ngroups, body, 0, unroll=False)
    plsc.subcore_barrier()
    pltpu.sync_copy(acc_sh.at[pl.ds(s * rpt, rpt)],
                    out_hbm.at[pl.ds(c * NP + s * rpt, rpt)])


def _sc_gat1_body(hw_hbm, ta_hbm, tb_hbm, isrc_hbm, idst_hbm, z_hbm, out_hbm,
                  isrc_v, idst_v, g1_v, g2_v, hw_v, rows_v, acc_sh, sem):
    c = lax.axis_index("c")
    s = lax.axis_index("s")
    wid = s * 2 + c
    rpt = NP // 16
    pltpu.sync_copy(z_hbm.at[pl.ds(s * rpt, rpt)], acc_sh.at[pl.ds(s * rpt, rpt)])
    pltpu.sync_copy(isrc_hbm.at[pl.ds(wid * GP, GP)], isrc_v)
    pltpu.sync_copy(idst_hbm.at[pl.ds(wid * GP, GP)], idst_v)
    plsc.subcore_barrier()

    def group(g, _):
        c1 = pltpu.async_copy(ta_hbm.at[isrc_v.at[g]], g1_v, sem)
        c2 = pltpu.async_copy(tb_hbm.at[idst_v.at[g]], g2_v, sem)
        c3 = pltpu.async_copy(hw_hbm.at[isrc_v.at[g]], hw_v, sem)
        c1.wait(); c2.wait(); c3.wait()

        def edge(i, _):
            asum = g1_v[i, :] + g2_v[i, :]
            al = jnp.where(asum >= 0.0, asum, 0.2 * asum)
            aexp = jnp.exp(al)
            rows_v[i, pl.ds(128, 16)] = aexp
            for j in range(H1):
                rows_v[i, pl.ds(j * 16, 16)] = hw_v[i, pl.ds(j * 16, 16)] * aexp
            return _

        lax.fori_loop(0, G, edge, 0, unroll=False)
        pltpu.sync_copy(rows_v, acc_sh.at[idst_v.at[g]], add=True)
        return _

    lax.fori_loop(0, GP, group, 0, unroll=False)
    plsc.subcore_barrier()
    pltpu.sync_copy(acc_sh.at[pl.ds(s * rpt, rpt)],
                    out_hbm.at[pl.ds(c * NP + s * rpt, rpt)])


def _sc_gat2_body(hw_hbm, ta_hbm, tb_hbm, isrc_hbm, idst_hbm, z_hbm, out_hbm,
                  isrc_v, idst_v, g1_v, g2_v, hw_v, rows_v, acc_sh, sem):
    c = lax.axis_index("c")
    s = lax.axis_index("s")
    wid = s * 2 + c
    rpt = NP // 16
    pltpu.sync_copy(z_hbm.at[pl.ds(s * rpt, rpt)], acc_sh.at[pl.ds(s * rpt, rpt)])
    pltpu.sync_copy(isrc_hbm.at[pl.ds(wid * GP, GP)], isrc_v)
    pltpu.sync_copy(idst_hbm.at[pl.ds(wid * GP, GP)], idst_v)
    plsc.subcore_barrier()

    def group(g, _):
        c1 = pltpu.async_copy(ta_hbm.at[isrc_v.at[g]], g1_v, sem)
        c2 = pltpu.async_copy(tb_hbm.at[idst_v.at[g]], g2_v, sem)
        c3 = pltpu.async_copy(hw_hbm.at[isrc_v.at[g]], hw_v, sem)
        c1.wait(); c2.wait(); c3.wait()

        def edge(i, _):
            asum = g1_v[i, :] + g2_v[i, :]
            al = jnp.where(asum >= 0.0, asum, 0.2 * asum)
            aexp = jnp.exp(al)
            rows_v[i, pl.ds(0, 16)] = hw_v[i, :] * aexp
            rows_v[i, pl.ds(16, 16)] = aexp
            return _

        lax.fori_loop(0, G, edge, 0, unroll=False)
        pltpu.sync_copy(rows_v, acc_sh.at[idst_v.at[g]], add=True)
        return _

    lax.fori_loop(0, GP, group, 0, unroll=False)
    plsc.subcore_barrier()
    pltpu.sync_copy(acc_sh.at[pl.ds(s * rpt, rpt)],
                    out_hbm.at[pl.ds(c * NP + s * rpt, rpt)])


def _tc_msg_body(ea_ref, xs_ref, w1_ref, b1_ref, w2r_ref, b2r_ref, out_ref):
    h1 = jnp.maximum(
        jnp.dot(ea_ref[...], w1_ref[...], preferred_element_type=jnp.float32)
        + b1_ref[...], 0.0)
    xs = xs_ref[...]
    a = jnp.concatenate([xs[:, f:f + 1] * h1 for f in range(F_IN)], axis=1)
    msg = jnp.dot(a, w2r_ref[...], preferred_element_type=jnp.float32)
    msg = msg + jnp.dot(xs, b2r_ref[...], preferred_element_type=jnp.float32)
    ones = jnp.ones((msg.shape[0], 1), jnp.float32)
    zer = jnp.zeros((msg.shape[0], W144 - HID - 1), jnp.float32)
    out_ref[...] = jnp.concatenate([msg, ones, zer], axis=1)


def _tc_h_gat1_body(acc_ref, x_ref, wroot_ref, broot_ref, w_ref, as_ref, ad_ref,
                    hw_ref, ta_ref, tb_ref):
    acc = acc_ref[0] + acc_ref[1]
    deg = jnp.maximum(acc[:, HID:HID + 1], 1.0)
    h = jnp.maximum(
        acc[:, :HID] / deg
        + jnp.dot(x_ref[...], wroot_ref[...], preferred_element_type=jnp.float32)
        + broot_ref[...], 0.0)
    hw = jnp.dot(h, w_ref[...], preferred_element_type=jnp.float32)
    hw_ref[...] = hw
    ta_ref[...] = jnp.dot(hw, as_ref[...], preferred_element_type=jnp.float32)
    tb_ref[...] = jnp.dot(hw, ad_ref[...], preferred_element_type=jnp.float32)


def _tc_gat2_prep_body(acc_ref, b_ref, w_ref, as_ref, ad_ref,
                       hw_ref, ta_ref, tb_ref):
    acc = acc_ref[0] + acc_ref[1]
    s = acc[:, :HID]
    d = acc[:, HID:HID + H1]
    dfull = jnp.tile(d, (1, C1))
    g1 = s / (dfull + 1e-16) + b_ref[...]
    h2 = jnp.where(g1 > 0.0, g1, jnp.exp(jnp.minimum(g1, 0.0)) - 1.0)
    hw = jnp.dot(h2, w_ref[...], preferred_element_type=jnp.float32)
    hw_ref[...] = hw
    ta_ref[...] = jnp.dot(hw, as_ref[...], preferred_element_type=jnp.float32)
    tb_ref[...] = jnp.dot(hw, ad_ref[...], preferred_element_type=jnp.float32)


def _tc_final_body(acc_ref, b_ref, out_ref):
    acc = acc_ref[0] + acc_ref[1]
    s = acc[:, :NCLS]
    d = acc[:, NCLS:NCLS + 1]
    o = s / (d + 1e-16) + b_ref[...]
    m = jnp.max(o, axis=1, keepdims=True)
    z = o - m
    lse = jnp.log(jnp.sum(jnp.exp(z), axis=1, keepdims=True))
    out_ref[...] = z - lse


def _sc_gather(tab, idx2d):
    k = functools.partial(
        pl.kernel,
        out_type=jax.ShapeDtypeStruct((EP, F_IN), jnp.float32),
        scratch_types=[
            pltpu.VMEM((GP, G), jnp.int32),
            pltpu.VMEM((2, G, F_IN), jnp.float32),
            pltpu.SemaphoreType.DMA,
        ],
        mesh=_mesh(),
        compiler_params=pltpu.CompilerParams(use_tc_tiling_on_sc=False),
    )(_sc_gather_body)
    return k(tab, idx2d)


def _sc_scatter(rows, idx2d, z):
    k = functools.partial(
        pl.kernel,
        out_type=jax.ShapeDtypeStruct((2 * NP, W144), jnp.float32),
        scratch_types=[
            pltpu.VMEM((GP, G), jnp.int32),
            pltpu.VMEM((G, W144), jnp.float32),
            pltpu.VMEM_SHARED((NP, W144), jnp.float32),
            pltpu.SemaphoreType.DMA,
        ],
        mesh=_mesh(),
        compiler_params=pltpu.CompilerParams(use_tc_tiling_on_sc=False),
    )(_sc_scatter_body)
    return k(rows, idx2d, z)


def _sc_gat(body, hw, ta, tb, isrc, idst, z, width):
    k = functools.partial(
        pl.kernel,
        out_type=jax.ShapeDtypeStruct((2 * NP, width), jnp.float32),
        scratch_types=[
            pltpu.VMEM((GP, G), jnp.int32),
            pltpu.VMEM((GP, G), jnp.int32),
            pltpu.VMEM((G, 16), jnp.float32),
            pltpu.VMEM((G, 16), jnp.float32),
            pltpu.VMEM((G, hw.shape[1]), jnp.float32),
            pltpu.VMEM((G, width), jnp.float32),
            pltpu.VMEM_SHARED((NP, width), jnp.float32),
            pltpu.SemaphoreType.DMA,
        ],
        mesh=_mesh(),
        compiler_params=pltpu.CompilerParams(use_tc_tiling_on_sc=False),
    )(body)
    return k(hw, ta, tb, isrc, idst, z)


def kernel(x, edge_index, edge_attr, W1, b1, W2, b2, Wroot, broot,
           gat1_W, gat1_asrc, gat1_adst, gat1_b,
           gat2_W, gat2_asrc, gat2_adst, gat2_b):
    f32 = jnp.float32
    src = edge_index[0]
    dst = edge_index[1]

    pad1 = EP - E
    src_p = jnp.concatenate([src, jnp.zeros((pad1,), src.dtype)])
    dst_p = jnp.concatenate([dst, jnp.full((pad1,), JUNK, dst.dtype)])
    ea_p = jnp.concatenate([edge_attr, jnp.zeros((pad1, F_IN), f32)])
    xp = jnp.concatenate([x, jnp.zeros((NP - N, F_IN), f32)])

    pad2 = EP - (E + N)
    loop = jnp.arange(N, dtype=src.dtype)
    src2 = jnp.concatenate([src, loop, jnp.zeros((pad2,), src.dtype)])
    dst2 = jnp.concatenate([dst, loop, jnp.full((pad2,), JUNK, dst.dtype)])

    W2r = W2.reshape(HID, F_IN, HID).transpose(1, 0, 2).reshape(F_IN * HID, HID)
    B2 = b2.reshape(F_IN, HID)
    b1r = b1.reshape(1, HID)
    brootr = broot.reshape(1, HID)

    g2br = gat2_b.reshape(1, NCLS)

    hh = jnp.arange(H1)
    As3 = jnp.zeros((H1, C1, H1), f32).at[hh, :, hh].set(gat1_asrc)
    Ad3 = jnp.zeros((H1, C1, H1), f32).at[hh, :, hh].set(gat1_adst)
    ASrep = jnp.concatenate([As3.reshape(HID, H1)] * 2, axis=1)
    ADrep = jnp.concatenate([Ad3.reshape(HID, H1)] * 2, axis=1)

    pidx = (jnp.arange(HID) % H1) * C1 + jnp.arange(HID) // H1
    gat1_Wp = gat1_W[:, pidx]
    ASrep_p = ASrep[pidx, :]
    ADrep_p = ADrep[pidx, :]
    g1b_p = gat1_b[pidx]
    gat2_Wp = gat2_W[pidx, :]
    As2rep = jnp.tile(gat2_asrc.reshape(NCLS, 1), (1, 16))
    Ad2rep = jnp.tile(gat2_adst.reshape(NCLS, 1), (1, 16))

    z144 = jnp.zeros((NP, W144), f32)
    z32 = jnp.zeros((NP, 32), f32)

    src_p2d = src_p.reshape(EP // G, G)
    dst_p2d = dst_p.reshape(EP // G, G)
    src2_2d = src2.reshape(EP // G, G)
    dst2_2d = dst2.reshape(EP // G, G)

    xs = _sc_gather(xp, src_p2d)

    nb = -(-E // BE)
    msg = pl.pallas_call(
        _tc_msg_body,
        grid=(nb,),
        in_specs=[
            pl.BlockSpec((BE, F_IN), lambda i: (i, 0)),
            pl.BlockSpec((BE, F_IN), lambda i: (i, 0)),
            pl.BlockSpec((F_IN, HID), lambda i: (0, 0)),
            pl.BlockSpec((1, HID), lambda i: (0, 0)),
            pl.BlockSpec((F_IN * HID, HID), lambda i: (0, 0)),
            pl.BlockSpec((F_IN, HID), lambda i: (0, 0)),
        ],
        out_specs=pl.BlockSpec((BE, W144), lambda i: (i, 0)),
        out_shape=jax.ShapeDtypeStruct((EP, W144), f32),
    )(ea_p, xs, W1, b1r, W2r, B2)

    acc1 = _sc_scatter(msg, dst_p2d, z144).reshape(2, NP, W144)

    nb3 = NP // 2512
    hw1, ta1, tb1 = pl.pallas_call(
        _tc_h_gat1_body,
        grid=(nb3,),
        in_specs=[
            pl.BlockSpec((2, 2512, W144), lambda i: (0, i, 0)),
            pl.BlockSpec((2512, F_IN), lambda i: (i, 0)),
            pl.BlockSpec((F_IN, HID), lambda i: (0, 0)),
            pl.BlockSpec((1, HID), lambda i: (0, 0)),
            pl.BlockSpec((HID, HID), lambda i: (0, 0)),
            pl.BlockSpec((HID, 16), lambda i: (0, 0)),
            pl.BlockSpec((HID, 16), lambda i: (0, 0)),
        ],
        out_specs=[
            pl.BlockSpec((2512, HID), lambda i: (i, 0)),
            pl.BlockSpec((2512, 16), lambda i: (i, 0)),
            pl.BlockSpec((2512, 16), lambda i: (i, 0)),
        ],
        out_shape=[
            jax.ShapeDtypeStruct((NP, HID), f32),
            jax.ShapeDtypeStruct((NP, 16), f32),
            jax.ShapeDtypeStruct((NP, 16), f32),
        ],
    )(acc1, xp, Wroot, brootr, gat1_Wp, ASrep_p, ADrep_p)

    acc2 = _sc_gat(_sc_gat1_body, hw1, ta1, tb1, src2_2d, dst2_2d, z144,
                   W144).reshape(2, NP, W144)

    hw2, ta2, tb2 = pl.pallas_call(
        _tc_gat2_prep_body,
        grid=(nb3,),
        in_specs=[
            pl.BlockSpec((2, 2512, W144), lambda i: (0, i, 0)),
            pl.BlockSpec((1, HID), lambda i: (0, 0)),
            pl.BlockSpec((HID, NCLS), lambda i: (0, 0)),
            pl.BlockSpec((NCLS, 16), lambda i: (0, 0)),
            pl.BlockSpec((NCLS, 16), lambda i: (0, 0)),
        ],
        out_specs=[
            pl.BlockSpec((2512, NCLS), lambda i: (i, 0)),
            pl.BlockSpec((2512, 16), lambda i: (i, 0)),
            pl.BlockSpec((2512, 16), lambda i: (i, 0)),
        ],
        out_shape=[
            jax.ShapeDtypeStruct((NP, NCLS), f32),
            jax.ShapeDtypeStruct((NP, 16), f32),
            jax.ShapeDtypeStruct((NP, 16), f32),
        ],
    )(acc2, g1b_p.reshape(1, H1 * C1), gat2_Wp, As2rep, Ad2rep)

    acc3 = _sc_gat(_sc_gat2_body, hw2, ta2, tb2, src2_2d, dst2_2d, z32,
                   32).reshape(2, NP, 32)

    out = pl.pallas_call(
        _tc_final_body,
        grid=(nb3,),
        in_specs=[
            pl.BlockSpec((2, 2512, 32), lambda i: (0, i, 0)),
            pl.BlockSpec((1, NCLS), lambda i: (0, 0)),
        ],
        out_specs=pl.BlockSpec((2512, NCLS), lambda i: (i, 0)),
        out_shape=jax.ShapeDtypeStruct((NP, NCLS), f32),
    )(acc3, g2br)

    return out[:N]


# speedup vs baseline: 15.0283x; 1.0378x over previous
import functools

import jax
import jax.numpy as jnp
from jax import lax
from jax.experimental import pallas as pl
from jax.experimental.pallas import tpu as pltpu
from jax.experimental.pallas import tpu_sc as plsc

N = 10000
E = 80000
F_IN = 16
HID = 128
H1, C1 = 8, 16
NCLS = 16

NP = 10048
JUNK = N
NWORK = 32
G = 96
EP = 98304
GP = EP // (NWORK * G)
BE = 512
W144 = 144

def _mesh():
    return plsc.VectorSubcoreMesh(core_axis_name="c", subcore_axis_name="s")


def _sc_gather_body(tab_hbm, idx2d_hbm, out_hbm, idx_v, rows_v, sem):
    c = lax.axis_index("c")
    s = lax.axis_index("s")
    wid = s * 2 + c
    ngroups = idx2d_hbm.shape[0] // NWORK
    pltpu.sync_copy(idx2d_hbm.at[pl.ds(wid * ngroups, ngroups)], idx_v)
    for p in range(3):
        pltpu.async_copy(tab_hbm.at[idx_v.at[p]], rows_v.at[p], sem)

    def body(g, _):
        b = lax.rem(g, 4)
        pltpu.make_async_copy(tab_hbm.at[idx_v.at[g]], rows_v.at[b], sem).wait()

        @pl.when(g + 3 < ngroups)
        def _pf():
            pltpu.async_copy(tab_hbm.at[idx_v.at[g + 3]],
                             rows_v.at[lax.rem(g + 3, 4)], sem)

        pltpu.sync_copy(rows_v.at[b],
                        out_hbm.at[pl.ds(wid * ngroups * G + g * G, G)])
        return _

    lax.fori_loop(0, ngroups, body, 0, unroll=False)


def _sc_scatter_body(rows_hbm, idx2d_hbm, z_hbm, out_hbm, idx_v, rows_v, acc_sh, sem, ssem):
    c = lax.axis_index("c")
    s = lax.axis_index("s")
    wid = s * 2 + c
    ngroups = idx2d_hbm.shape[0] // NWORK
    rpt = NP // 16
    pltpu.sync_copy(z_hbm.at[pl.ds(s * rpt, rpt)], acc_sh.at[pl.ds(s * rpt, rpt)])
    pltpu.sync_copy(idx2d_hbm.at[pl.ds(wid * ngroups, ngroups)], idx_v)
    plsc.subcore_barrier()

    pltpu.async_copy(rows_hbm.at[pl.ds(wid * ngroups * G, G)], rows_v.at[0], sem)

    def body(g, _):
        b = lax.rem(g, 2)
        pltpu.make_async_copy(
            rows_hbm.at[pl.ds(wid * ngroups * G + g * G, G)], rows_v.at[b],
            sem).wait()

        @pl.when(g >= 1)
        def _ws():
            pltpu.make_async_copy(rows_v.at[1 - b], acc_sh.at[idx_v.at[g - 1]],
                                  ssem).wait()

        @pl.when(g + 1 < ngroups)
        def _pf():
            pltpu.async_copy(
                rows_hbm.at[pl.ds(wid * ngroups * G + (g + 1) * G, G)],
                rows_v.at[1 - b], sem)

        pltpu.async_copy(rows_v.at[b], acc_sh.at[idx_v.at[g]], ssem, add=True)
        return _

    lax.fori_loop(0, ngroups, body, 0, unroll=False)
    pltpu.make_async_copy(rows_v.at[lax.rem(ngroups - 1, 2)],
                          acc_sh.at[idx_v.at[ngroups - 1]], ssem).wait()
    plsc.subcore_barrier()
    pltpu.sync_copy(acc_sh.at[pl.ds(s * rpt, rpt)],
                    out_hbm.at[pl.ds(c * NP + s * rpt, rpt)])


def _sc_gat1_body(thw_hbm, tb_hbm, isrc_hbm, idst_hbm, z_hbm, out_hbm,
                  isrc_v, idst_v, tb_v, rows_v, acc_sh, gsem, ssem):
    c = lax.axis_index("c")
    s = lax.axis_index("s")
    wid = s * 2 + c
    rpt = NP // 16
    pltpu.sync_copy(z_hbm.at[pl.ds(s * rpt, rpt)], acc_sh.at[pl.ds(s * rpt, rpt)])
    pltpu.sync_copy(isrc_hbm.at[pl.ds(wid * GP, GP)], isrc_v)
    pltpu.sync_copy(idst_hbm.at[pl.ds(wid * GP, GP)], idst_v)
    plsc.subcore_barrier()
    pltpu.async_copy(thw_hbm.at[isrc_v.at[0]], rows_v.at[0], gsem)
    pltpu.async_copy(tb_hbm.at[idst_v.at[0]], tb_v.at[0], gsem)

    def group(g, _):
        b = lax.rem(g, 2)
        pltpu.make_async_copy(thw_hbm.at[isrc_v.at[g]], rows_v.at[b], gsem).wait()
        pltpu.make_async_copy(tb_hbm.at[idst_v.at[g]], tb_v.at[b], gsem).wait()

        @pl.when(g >= 1)
        def _ws():
            pltpu.make_async_copy(rows_v.at[1 - b], acc_sh.at[idst_v.at[g - 1]],
                                  ssem).wait()

        @pl.when(g + 1 < GP)
        def _pf():
            pltpu.async_copy(thw_hbm.at[isrc_v.at[g + 1]], rows_v.at[1 - b], gsem)
            pltpu.async_copy(tb_hbm.at[idst_v.at[g + 1]], tb_v.at[1 - b], gsem)

        def edge(i, _):
            asum = rows_v[b, i, pl.ds(128, 16)] + tb_v[b, i, :]
            al = jnp.where(asum >= 0.0, asum, 0.2 * asum)
            aexp = jnp.exp(al)
            for j in range(H1):
                rows_v[b, i, pl.ds(j * 16, 16)] = (
                    rows_v[b, i, pl.ds(j * 16, 16)] * aexp)
            rows_v[b, i, pl.ds(128, 16)] = aexp
            return _

        lax.fori_loop(0, G, edge, 0, unroll=False)
        pltpu.async_copy(rows_v.at[b], acc_sh.at[idst_v.at[g]], ssem, add=True)
        return _

    lax.fori_loop(0, GP, group, 0, unroll=False)
    pltpu.make_async_copy(rows_v.at[lax.rem(GP - 1, 2)],
                          acc_sh.at[idst_v.at[GP - 1]], ssem).wait()
    plsc.subcore_barrier()
    pltpu.sync_copy(acc_sh.at[pl.ds(s * rpt, rpt)],
                    out_hbm.at[pl.ds(c * NP + s * rpt, rpt)])


def _sc_gat2_body(thw_hbm, tb_hbm, isrc_hbm, idst_hbm, z_hbm, out_hbm,
                  isrc_v, idst_v, tb_v, rows_v, acc_sh, gsem, ssem):
    c = lax.axis_index("c")
    s = lax.axis_index("s")
    wid = s * 2 + c
    rpt = NP // 16
    pltpu.sync_copy(z_hbm.at[pl.ds(s * rpt, rpt)], acc_sh.at[pl.ds(s * rpt, rpt)])
    pltpu.sync_copy(isrc_hbm.at[pl.ds(wid * GP, GP)], isrc_v)
    pltpu.sync_copy(idst_hbm.at[pl.ds(wid * GP, GP)], idst_v)
    plsc.subcore_barrier()
    pltpu.async_copy(thw_hbm.at[isrc_v.at[0]], rows_v.at[0], gsem)
    pltpu.async_copy(tb_hbm.at[idst_v.at[0]], tb_v.at[0], gsem)

    def group(g, _):
        b = lax.rem(g, 2)
        pltpu.make_async_copy(thw_hbm.at[isrc_v.at[g]], rows_v.at[b], gsem).wait()
        pltpu.make_async_copy(tb_hbm.at[idst_v.at[g]], tb_v.at[b], gsem).wait()

        @pl.when(g >= 1)
        def _ws():
            pltpu.make_async_copy(rows_v.at[1 - b], acc_sh.at[idst_v.at[g - 1]],
                                  ssem).wait()

        @pl.when(g + 1 < GP)
        def _pf():
            pltpu.async_copy(thw_hbm.at[isrc_v.at[g + 1]], rows_v.at[1 - b], gsem)
            pltpu.async_copy(tb_hbm.at[idst_v.at[g + 1]], tb_v.at[1 - b], gsem)

        def edge(i, _):
            asum = rows_v[b, i, pl.ds(16, 16)] + tb_v[b, i, :]
            al = jnp.where(asum >= 0.0, asum, 0.2 * asum)
            aexp = jnp.exp(al)
            rows_v[b, i, pl.ds(0, 16)] = rows_v[b, i, pl.ds(0, 16)] * aexp
            rows_v[b, i, pl.ds(16, 16)] = aexp
            return _

        lax.fori_loop(0, G, edge, 0, unroll=False)
        pltpu.async_copy(rows_v.at[b], acc_sh.at[idst_v.at[g]], ssem, add=True)
        return _

    lax.fori_loop(0, GP, group, 0, unroll=False)
    pltpu.make_async_copy(rows_v.at[lax.rem(GP - 1, 2)],
                          acc_sh.at[idst_v.at[GP - 1]], ssem).wait()
    plsc.subcore_barrier()
    pltpu.sync_copy(acc_sh.at[pl.ds(s * rpt, rpt)],
                    out_hbm.at[pl.ds(c * NP + s * rpt, rpt)])


def _tc_msg_body(ea_ref, xs_ref, w1_ref, b1_ref, w2r_ref, b2r_ref, out_ref):
    h1 = jnp.maximum(
        jnp.dot(ea_ref[...], w1_ref[...], preferred_element_type=jnp.float32)
        + b1_ref[...], 0.0)
    xs = xs_ref[...]
    a = jnp.concatenate([xs[:, f:f + 1] * h1 for f in range(F_IN)], axis=1)
    msg = jnp.dot(a, w2r_ref[...], preferred_element_type=jnp.float32)
    msg = msg + jnp.dot(xs, b2r_ref[...], preferred_element_type=jnp.float32)
    ones = jnp.ones((msg.shape[0], 1), jnp.float32)
    zer = jnp.zeros((msg.shape[0], W144 - HID - 1), jnp.float32)
    out_ref[...] = jnp.concatenate([msg, ones, zer], axis=1)


def _tc_h_gat1_body(acc_ref, x_ref, wroot_ref, broot_ref, w_ref, as_ref, ad_ref,
                    thw_ref, tb_ref):
    acc = acc_ref[0] + acc_ref[1]
    deg = jnp.maximum(acc[:, HID:HID + 1], 1.0)
    h = jnp.maximum(
        acc[:, :HID] / deg
        + jnp.dot(x_ref[...], wroot_ref[...], preferred_element_type=jnp.float32)
        + broot_ref[...], 0.0)
    hw = jnp.dot(h, w_ref[...], preferred_element_type=jnp.float32)
    ta = jnp.dot(hw, as_ref[...], preferred_element_type=jnp.float32)
    thw_ref[...] = jnp.concatenate([hw, ta], axis=1)
    tb_ref[...] = jnp.dot(hw, ad_ref[...], preferred_element_type=jnp.float32)


def _tc_gat2_prep_body(acc_ref, b_ref, w_ref, as_ref, ad_ref,
                       thw_ref, tb_ref):
    acc = acc_ref[0] + acc_ref[1]
    s = acc[:, :HID]
    d = acc[:, HID:HID + H1]
    dfull = jnp.tile(d, (1, C1))
    g1 = s / (dfull + 1e-16) + b_ref[...]
    h2 = jnp.where(g1 > 0.0, g1, jnp.exp(jnp.minimum(g1, 0.0)) - 1.0)
    hw = jnp.dot(h2, w_ref[...], preferred_element_type=jnp.float32)
    ta = jnp.dot(hw, as_ref[...], preferred_element_type=jnp.float32)
    thw_ref[...] = jnp.concatenate([hw, ta], axis=1)
    tb_ref[...] = jnp.dot(hw, ad_ref[...], preferred_element_type=jnp.float32)


def _tc_final_body(acc_ref, b_ref, out_ref):
    acc = acc_ref[0] + acc_ref[1]
    s = acc[:, :NCLS]
    d = acc[:, NCLS:NCLS + 1]
    o = s / (d + 1e-16) + b_ref[...]
    m = jnp.max(o, axis=1, keepdims=True)
    z = o - m
    lse = jnp.log(jnp.sum(jnp.exp(z), axis=1, keepdims=True))
    out_ref[...] = z - lse


def _sc_gather(tab, idx2d):
    k = functools.partial(
        pl.kernel,
        out_type=jax.ShapeDtypeStruct((EP, F_IN), jnp.float32),
        scratch_types=[
            pltpu.VMEM((GP, G), jnp.int32),
            pltpu.VMEM((2, G, F_IN), jnp.float32),
            pltpu.SemaphoreType.DMA,
        ],
        mesh=_mesh(),
        compiler_params=pltpu.CompilerParams(use_tc_tiling_on_sc=False),
    )(_sc_gather_body)
    return k(tab, idx2d)


def _sc_scatter(rows, idx2d, z):
    k = functools.partial(
        pl.kernel,
        out_type=jax.ShapeDtypeStruct((2 * NP, W144), jnp.float32),
        scratch_types=[
            pltpu.VMEM((GP, G), jnp.int32),
            pltpu.VMEM((2, G, W144), jnp.float32),
            pltpu.VMEM_SHARED((NP, W144), jnp.float32),
            pltpu.SemaphoreType.DMA,
            pltpu.SemaphoreType.DMA,
        ],
        mesh=_mesh(),
        compiler_params=pltpu.CompilerParams(use_tc_tiling_on_sc=False),
    )(_sc_scatter_body)
    return k(rows, idx2d, z)


def _sc_gat(body, thw, tb, isrc, idst, z, width):
    k = functools.partial(
        pl.kernel,
        out_type=jax.ShapeDtypeStruct((2 * NP, width), jnp.float32),
        scratch_types=[
            pltpu.VMEM((GP, G), jnp.int32),
            pltpu.VMEM((GP, G), jnp.int32),
            pltpu.VMEM((2, G, 16), jnp.float32),
            pltpu.VMEM((2, G, width), jnp.float32),
            pltpu.VMEM_SHARED((NP, width), jnp.float32),
            pltpu.SemaphoreType.DMA,
            pltpu.SemaphoreType.DMA,
        ],
        mesh=_mesh(),
        compiler_params=pltpu.CompilerParams(use_tc_tiling_on_sc=False),
    )(body)
    return k(thw, tb, isrc, idst, z)


def kernel(x, edge_index, edge_attr, W1, b1, W2, b2, Wroot, broot,
           gat1_W, gat1_asrc, gat1_adst, gat1_b,
           gat2_W, gat2_asrc, gat2_adst, gat2_b):
    f32 = jnp.float32
    src = edge_index[0]
    dst = edge_index[1]

    pad1 = EP - E
    src_p = jnp.concatenate([src, jnp.zeros((pad1,), src.dtype)])
    dst_p = jnp.concatenate([dst, jnp.full((pad1,), JUNK, dst.dtype)])
    ea_p = jnp.concatenate([edge_attr, jnp.zeros((pad1, F_IN), f32)])
    xp = jnp.concatenate([x, jnp.zeros((NP - N, F_IN), f32)])

    pad2 = EP - (E + N)
    loop = jnp.arange(N, dtype=src.dtype)
    src2 = jnp.concatenate([src, loop, jnp.zeros((pad2,), src.dtype)])
    dst2 = jnp.concatenate([dst, loop, jnp.full((pad2,), JUNK, dst.dtype)])

    W2r = W2.reshape(HID, F_IN, HID).transpose(1, 0, 2).reshape(F_IN * HID, HID)
    B2 = b2.reshape(F_IN, HID)
    b1r = b1.reshape(1, HID)
    brootr = broot.reshape(1, HID)

    g2br = gat2_b.reshape(1, NCLS)

    hh = jnp.arange(H1)
    As3 = jnp.zeros((H1, C1, H1), f32).at[hh, :, hh].set(gat1_asrc)
    Ad3 = jnp.zeros((H1, C1, H1), f32).at[hh, :, hh].set(gat1_adst)
    ASrep = jnp.concatenate([As3.reshape(HID, H1)] * 2, axis=1)
    ADrep = jnp.concatenate([Ad3.reshape(HID, H1)] * 2, axis=1)

    pidx = (jnp.arange(HID) % H1) * C1 + jnp.arange(HID) // H1
    gat1_Wp = gat1_W[:, pidx]
    ASrep_p = ASrep[pidx, :]
    ADrep_p = ADrep[pidx, :]
    g1b_p = gat1_b[pidx]
    gat2_Wp = gat2_W[pidx, :]
    As2rep = jnp.tile(gat2_asrc.reshape(NCLS, 1), (1, 16))
    Ad2rep = jnp.tile(gat2_adst.reshape(NCLS, 1), (1, 16))

    z144 = jnp.zeros((NP, W144), f32)
    z32 = jnp.zeros((NP, 32), f32)

    src_p2d = src_p.reshape(EP // G, G)
    dst_p2d = dst_p.reshape(EP // G, G)
    src2_2d = src2.reshape(EP // G, G)
    dst2_2d = dst2.reshape(EP // G, G)

    xs = _sc_gather(xp, src_p2d)

    nb = -(-E // BE)
    msg = pl.pallas_call(
        _tc_msg_body,
        grid=(nb,),
        in_specs=[
            pl.BlockSpec((BE, F_IN), lambda i: (i, 0)),
            pl.BlockSpec((BE, F_IN), lambda i: (i, 0)),
            pl.BlockSpec((F_IN, HID), lambda i: (0, 0)),
            pl.BlockSpec((1, HID), lambda i: (0, 0)),
            pl.BlockSpec((F_IN * HID, HID), lambda i: (0, 0)),
            pl.BlockSpec((F_IN, HID), lambda i: (0, 0)),
        ],
        out_specs=pl.BlockSpec((BE, W144), lambda i: (i, 0)),
        out_shape=jax.ShapeDtypeStruct((EP, W144), f32),
    )(ea_p, xs, W1, b1r, W2r, B2)

    acc1 = _sc_scatter(msg, dst_p2d, z144).reshape(2, NP, W144)

    nb3 = NP // 2512
    thw1, tb1 = pl.pallas_call(
        _tc_h_gat1_body,
        grid=(nb3,),
        in_specs=[
            pl.BlockSpec((2, 2512, W144), lambda i: (0, i, 0)),
            pl.BlockSpec((2512, F_IN), lambda i: (i, 0)),
            pl.BlockSpec((F_IN, HID), lambda i: (0, 0)),
            pl.BlockSpec((1, HID), lambda i: (0, 0)),
            pl.BlockSpec((HID, HID), lambda i: (0, 0)),
            pl.BlockSpec((HID, 16), lambda i: (0, 0)),
            pl.BlockSpec((HID, 16), lambda i: (0, 0)),
        ],
        out_specs=[
            pl.BlockSpec((2512, W144), lambda i: (i, 0)),
            pl.BlockSpec((2512, 16), lambda i: (i, 0)),
        ],
        out_shape=[
            jax.ShapeDtypeStruct((NP, W144), f32),
            jax.ShapeDtypeStruct((NP, 16), f32),
        ],
    )(acc1, xp, Wroot, brootr, gat1_Wp, ASrep_p, ADrep_p)

    acc2 = _sc_gat(_sc_gat1_body, thw1, tb1, src2_2d, dst2_2d, z144,
                   W144).reshape(2, NP, W144)

    thw2, tb2 = pl.pallas_call(
        _tc_gat2_prep_body,
        grid=(nb3,),
        in_specs=[
            pl.BlockSpec((2, 2512, W144), lambda i: (0, i, 0)),
            pl.BlockSpec((1, HID), lambda i: (0, 0)),
            pl.BlockSpec((HID, NCLS), lambda i: (0, 0)),
            pl.BlockSpec((NCLS, 16), lambda i: (0, 0)),
            pl.BlockSpec((NCLS, 16), lambda i: (0, 0)),
        ],
        out_specs=[
            pl.BlockSpec((2512, 32), lambda i: (i, 0)),
            pl.BlockSpec((2512, 16), lambda i: (i, 0)),
        ],
        out_shape=[
            jax.ShapeDtypeStruct((NP, 32), f32),
            jax.ShapeDtypeStruct((NP, 16), f32),
        ],
    )(acc2, g1b_p.reshape(1, H1 * C1), gat2_Wp, As2rep, Ad2rep)

    acc3 = _sc_gat(_sc_gat2_body, thw2, tb2, src2_2d, dst2_2d, z32,
                   32).reshape(2, NP, 32)

    out = pl.pallas_call(
        _tc_final_body,
        grid=(nb3,),
        in_specs=[
            pl.BlockSpec((2, 2512, 32), lambda i: (0, i, 0)),
            pl.BlockSpec((1, NCLS), lambda i: (0, 0)),
        ],
        out_specs=pl.BlockSpec((2512, NCLS), lambda i: (i, 0)),
        out_shape=jax.ShapeDtypeStruct((NP, NCLS), f32),
    )(acc3, g2br)

    return out[:N]


# speedup vs baseline: 15.5005x; 1.0314x over previous
import functools

import jax
import jax.numpy as jnp
from jax import lax
from jax.experimental import pallas as pl
from jax.experimental.pallas import tpu as pltpu
from jax.experimental.pallas import tpu_sc as plsc

N = 10000
E = 80000
F_IN = 16
HID = 128
H1, C1 = 8, 16
NCLS = 16

NP = 10048
JUNK = N
NWORK = 32
G = 96
EP = 98304
GP = EP // (NWORK * G)
BE = 512
W144 = 144

def _mesh():
    return plsc.VectorSubcoreMesh(core_axis_name="c", subcore_axis_name="s")


def _sc_gather_body(tab_hbm, idx2d_hbm, out_hbm, idx_v, rows_v, sem):
    c = lax.axis_index("c")
    s = lax.axis_index("s")
    wid = s * 2 + c
    ngroups = idx2d_hbm.shape[0] // NWORK
    pltpu.sync_copy(idx2d_hbm.at[pl.ds(wid * ngroups, ngroups)], idx_v)
    for p in range(3):
        pltpu.async_copy(tab_hbm.at[idx_v.at[p]], rows_v.at[p], sem)

    def body(g, _):
        b = lax.rem(g, 4)
        pltpu.make_async_copy(tab_hbm.at[idx_v.at[g]], rows_v.at[b], sem).wait()

        @pl.when(g + 3 < ngroups)
        def _pf():
            pltpu.async_copy(tab_hbm.at[idx_v.at[g + 3]],
                             rows_v.at[lax.rem(g + 3, 4)], sem)

        pltpu.sync_copy(rows_v.at[b],
                        out_hbm.at[pl.ds(wid * ngroups * G + g * G, G)])
        return _

    lax.fori_loop(0, ngroups, body, 0, unroll=False)


def _sc_scatter_body(rows_hbm, idx2d_hbm, z_hbm, out_hbm, idx_v, rows_v, acc_sh, sem, ssem):
    c = lax.axis_index("c")
    s = lax.axis_index("s")
    wid = s * 2 + c
    ngroups = idx2d_hbm.shape[0] // NWORK
    rpt = NP // 16
    pltpu.sync_copy(z_hbm.at[pl.ds(s * rpt, rpt)], acc_sh.at[pl.ds(s * rpt, rpt)])
    pltpu.sync_copy(idx2d_hbm.at[pl.ds(wid * ngroups, ngroups)], idx_v)
    plsc.subcore_barrier()

    pltpu.async_copy(rows_hbm.at[pl.ds(wid * ngroups * G, G)], rows_v.at[0], sem)

    def body(g, _):
        b = lax.rem(g, 2)
        pltpu.make_async_copy(
            rows_hbm.at[pl.ds(wid * ngroups * G + g * G, G)], rows_v.at[b],
            sem).wait()

        @pl.when(g >= 1)
        def _ws():
            pltpu.make_async_copy(rows_v.at[1 - b], acc_sh.at[idx_v.at[g - 1]],
                                  ssem).wait()

        @pl.when(g + 1 < ngroups)
        def _pf():
            pltpu.async_copy(
                rows_hbm.at[pl.ds(wid * ngroups * G + (g + 1) * G, G)],
                rows_v.at[1 - b], sem)

        pltpu.async_copy(rows_v.at[b], acc_sh.at[idx_v.at[g]], ssem, add=True)
        return _

    lax.fori_loop(0, ngroups, body, 0, unroll=False)
    pltpu.make_async_copy(rows_v.at[lax.rem(ngroups - 1, 2)],
                          acc_sh.at[idx_v.at[ngroups - 1]], ssem).wait()
    plsc.subcore_barrier()
    pltpu.sync_copy(acc_sh.at[pl.ds(s * rpt, rpt)],
                    out_hbm.at[pl.ds(c * NP + s * rpt, rpt)])


def _sc_gat1_body(thw_hbm, tb_hbm, isrc_hbm, idst_hbm, z_hbm, out_hbm,
                  isrc_v, idst_v, tb_v, rows_v, acc_sh, gsem, ssem):
    c = lax.axis_index("c")
    s = lax.axis_index("s")
    wid = s * 2 + c
    rpt = NP // 16
    pltpu.sync_copy(z_hbm.at[pl.ds(s * rpt, rpt)], acc_sh.at[pl.ds(s * rpt, rpt)])
    pltpu.sync_copy(isrc_hbm.at[pl.ds(wid * GP, GP)], isrc_v)
    pltpu.sync_copy(idst_hbm.at[pl.ds(wid * GP, GP)], idst_v)
    plsc.subcore_barrier()
    pltpu.async_copy(thw_hbm.at[isrc_v.at[0]], rows_v.at[0], gsem)
    pltpu.async_copy(tb_hbm.at[idst_v.at[0]], tb_v.at[0], gsem)

    def group(g, _):
        b = lax.rem(g, 2)
        pltpu.make_async_copy(thw_hbm.at[isrc_v.at[g]], rows_v.at[b], gsem).wait()
        pltpu.make_async_copy(tb_hbm.at[idst_v.at[g]], tb_v.at[b], gsem).wait()

        @pl.when(g >= 1)
        def _ws():
            pltpu.make_async_copy(rows_v.at[1 - b], acc_sh.at[idst_v.at[g - 1]],
                                  ssem).wait()

        @pl.when(g + 1 < GP)
        def _pf():
            pltpu.async_copy(thw_hbm.at[isrc_v.at[g + 1]], rows_v.at[1 - b], gsem)
            pltpu.async_copy(tb_hbm.at[idst_v.at[g + 1]], tb_v.at[1 - b], gsem)

        def edge(i, _):
            asum = rows_v[b, i, pl.ds(128, 16)] + tb_v[b, i, :]
            al = jnp.where(asum >= 0.0, asum, 0.2 * asum)
            aexp = jnp.exp(al)
            for j in range(H1):
                rows_v[b, i, pl.ds(j * 16, 16)] = (
                    rows_v[b, i, pl.ds(j * 16, 16)] * aexp)
            rows_v[b, i, pl.ds(128, 16)] = aexp
            return _

        lax.fori_loop(0, G, edge, 0, unroll=False)
        pltpu.async_copy(rows_v.at[b], acc_sh.at[idst_v.at[g]], ssem, add=True)
        return _

    lax.fori_loop(0, GP, group, 0, unroll=False)
    pltpu.make_async_copy(rows_v.at[lax.rem(GP - 1, 2)],
                          acc_sh.at[idst_v.at[GP - 1]], ssem).wait()
    plsc.subcore_barrier()
    pltpu.sync_copy(acc_sh.at[pl.ds(s * rpt, rpt)],
                    out_hbm.at[pl.ds(c * NP + s * rpt, rpt)])


def _sc_gat2_body(thw_hbm, tb_hbm, isrc_hbm, idst_hbm, z_hbm, out_hbm,
                  isrc_v, idst_v, tb_v, rows_v, acc_sh, gsem, ssem):
    c = lax.axis_index("c")
    s = lax.axis_index("s")
    wid = s * 2 + c
    rpt = NP // 16
    pltpu.sync_copy(z_hbm.at[pl.ds(s * rpt, rpt)], acc_sh.at[pl.ds(s * rpt, rpt)])
    pltpu.sync_copy(isrc_hbm.at[pl.ds(wid * GP, GP)], isrc_v)
    pltpu.sync_copy(idst_hbm.at[pl.ds(wid * GP, GP)], idst_v)
    plsc.subcore_barrier()
    pltpu.async_copy(thw_hbm.at[isrc_v.at[0]], rows_v.at[0], gsem)
    pltpu.async_copy(tb_hbm.at[idst_v.at[0]], tb_v.at[0], gsem)

    def group(g, _):
        b = lax.rem(g, 2)
        pltpu.make_async_copy(thw_hbm.at[isrc_v.at[g]], rows_v.at[b], gsem).wait()
        pltpu.make_async_copy(tb_hbm.at[idst_v.at[g]], tb_v.at[b], gsem).wait()

        @pl.when(g >= 1)
        def _ws():
            pltpu.make_async_copy(rows_v.at[1 - b], acc_sh.at[idst_v.at[g - 1]],
                                  ssem).wait()

        @pl.when(g + 1 < GP)
        def _pf():
            pltpu.async_copy(thw_hbm.at[isrc_v.at[g + 1]], rows_v.at[1 - b], gsem)
            pltpu.async_copy(tb_hbm.at[idst_v.at[g + 1]], tb_v.at[1 - b], gsem)

        def edge(i, _):
            asum = rows_v[b, i, pl.ds(16, 16)] + tb_v[b, i, :]
            al = jnp.where(asum >= 0.0, asum, 0.2 * asum)
            aexp = jnp.exp(al)
            rows_v[b, i, pl.ds(0, 16)] = rows_v[b, i, pl.ds(0, 16)] * aexp
            rows_v[b, i, pl.ds(16, 16)] = aexp
            return _

        lax.fori_loop(0, G, edge, 0, unroll=False)
        pltpu.async_copy(rows_v.at[b], acc_sh.at[idst_v.at[g]], ssem, add=True)
        return _

    lax.fori_loop(0, GP, group, 0, unroll=False)
    pltpu.make_async_copy(rows_v.at[lax.rem(GP - 1, 2)],
                          acc_sh.at[idst_v.at[GP - 1]], ssem).wait()
    plsc.subcore_barrier()
    pltpu.sync_copy(acc_sh.at[pl.ds(s * rpt, rpt)],
                    out_hbm.at[pl.ds(c * NP + s * rpt, rpt)])


def _tc_msg_body(ea_ref, xs_ref, w1_ref, b1_ref, w2r_ref, b2r_ref, out_ref):
    h1 = jnp.maximum(
        jnp.dot(ea_ref[...], w1_ref[...], preferred_element_type=jnp.float32)
        + b1_ref[...], 0.0)
    xs = xs_ref[...]
    a = jnp.concatenate([xs[:, f:f + 1] * h1 for f in range(F_IN)], axis=1)
    msg = jnp.dot(a, w2r_ref[...], preferred_element_type=jnp.float32)
    msg = msg + jnp.dot(xs, b2r_ref[...], preferred_element_type=jnp.float32)
    ones = jnp.ones((msg.shape[0], 1), jnp.float32)
    zer = jnp.zeros((msg.shape[0], W144 - HID - 1), jnp.float32)
    out_ref[...] = jnp.concatenate([msg, ones, zer], axis=1)


def _tc_h_gat1_body(acc_ref, x_ref, wroot_ref, broot_ref, w_ref, as_ref, ad_ref,
                    thw_ref, tb_ref):
    acc = acc_ref[0] + acc_ref[1]
    deg = jnp.maximum(acc[:, HID:HID + 1], 1.0)
    h = jnp.maximum(
        acc[:, :HID] / deg
        + jnp.dot(x_ref[...], wroot_ref[...], preferred_element_type=jnp.float32)
        + broot_ref[...], 0.0)
    hw = jnp.dot(h, w_ref[...], preferred_element_type=jnp.float32)
    ta = jnp.dot(hw, as_ref[...], preferred_element_type=jnp.float32)
    thw_ref[...] = jnp.concatenate([hw, ta], axis=1)
    tb_ref[...] = jnp.dot(hw, ad_ref[...], preferred_element_type=jnp.float32)


def _tc_gat2_prep_body(acc_ref, b_ref, w_ref, as_ref, ad_ref,
                       thw_ref, tb_ref):
    acc = acc_ref[0] + acc_ref[1]
    s = acc[:, :HID]
    d = acc[:, HID:HID + H1]
    dfull = jnp.tile(d, (1, C1))
    g1 = s / (dfull + 1e-16) + b_ref[...]
    h2 = jnp.where(g1 > 0.0, g1, jnp.exp(jnp.minimum(g1, 0.0)) - 1.0)
    hw = jnp.dot(h2, w_ref[...], preferred_element_type=jnp.float32)
    ta = jnp.dot(hw, as_ref[...], preferred_element_type=jnp.float32)
    thw_ref[...] = jnp.concatenate([hw, ta], axis=1)
    tb_ref[...] = jnp.dot(hw, ad_ref[...], preferred_element_type=jnp.float32)


def _tc_final_body(acc_ref, b_ref, out_ref):
    acc = acc_ref[0] + acc_ref[1]
    s = acc[:, :NCLS]
    d = acc[:, NCLS:NCLS + 1]
    o = s / (d + 1e-16) + b_ref[...]
    m = jnp.max(o, axis=1, keepdims=True)
    z = o - m
    lse = jnp.log(jnp.sum(jnp.exp(z), axis=1, keepdims=True))
    out_ref[...] = z - lse


def _sc_gather(tab, idx2d):
    k = functools.partial(
        pl.kernel,
        out_type=jax.ShapeDtypeStruct((EP, F_IN), jnp.float32),
        scratch_types=[
            pltpu.VMEM((GP, G), jnp.int32),
            pltpu.VMEM((2, G, F_IN), jnp.float32),
            pltpu.SemaphoreType.DMA,
        ],
        mesh=_mesh(),
        compiler_params=pltpu.CompilerParams(use_tc_tiling_on_sc=False),
    )(_sc_gather_body)
    return k(tab, idx2d)


def _sc_scatter(rows, idx2d, z):
    k = functools.partial(
        pl.kernel,
        out_type=jax.ShapeDtypeStruct((2 * NP, W144), jnp.float32),
        scratch_types=[
            pltpu.VMEM((GP, G), jnp.int32),
            pltpu.VMEM((2, G, W144), jnp.float32),
            pltpu.VMEM_SHARED((NP, W144), jnp.float32),
            pltpu.SemaphoreType.DMA,
            pltpu.SemaphoreType.DMA,
        ],
        mesh=_mesh(),
        compiler_params=pltpu.CompilerParams(use_tc_tiling_on_sc=False),
    )(_sc_scatter_body)
    return k(rows, idx2d, z)


def _sc_gat(body, thw, tb, isrc, idst, z, width):
    k = functools.partial(
        pl.kernel,
        out_type=jax.ShapeDtypeStruct((2 * NP, width), jnp.float32),
        scratch_types=[
            pltpu.VMEM((GP, G), jnp.int32),
            pltpu.VMEM((GP, G), jnp.int32),
            pltpu.VMEM((2, G, 16), jnp.float32),
            pltpu.VMEM((2, G, width), jnp.float32),
            pltpu.VMEM_SHARED((NP, width), jnp.float32),
            pltpu.SemaphoreType.DMA,
            pltpu.SemaphoreType.DMA,
        ],
        mesh=_mesh(),
        compiler_params=pltpu.CompilerParams(use_tc_tiling_on_sc=False),
    )(body)
    return k(thw, tb, isrc, idst, z)


def kernel(x, edge_index, edge_attr, W1, b1, W2, b2, Wroot, broot,
           gat1_W, gat1_asrc, gat1_adst, gat1_b,
           gat2_W, gat2_asrc, gat2_adst, gat2_b):
    f32 = jnp.float32
    src = edge_index[0]
    dst = edge_index[1]

    pad1 = EP - E
    src_p = jnp.concatenate([src, jnp.zeros((pad1,), src.dtype)])
    dst_p = jnp.concatenate([dst, jnp.full((pad1,), JUNK, dst.dtype)])
    xp = jnp.concatenate([x, jnp.zeros((NP - N, F_IN), f32)])

    pad2 = EP - (E + N)
    loop = jnp.arange(N, dtype=src.dtype)
    src2 = jnp.concatenate([src, loop, jnp.zeros((pad2,), src.dtype)])
    dst2 = jnp.concatenate([dst, loop, jnp.full((pad2,), JUNK, dst.dtype)])

    W2r = W2.reshape(HID, F_IN, HID).transpose(1, 0, 2).reshape(F_IN * HID, HID)
    B2 = b2.reshape(F_IN, HID)
    b1r = b1.reshape(1, HID)
    brootr = broot.reshape(1, HID)

    g2br = gat2_b.reshape(1, NCLS)

    hh = jnp.arange(H1)
    As3 = jnp.zeros((H1, C1, H1), f32).at[hh, :, hh].set(gat1_asrc)
    Ad3 = jnp.zeros((H1, C1, H1), f32).at[hh, :, hh].set(gat1_adst)
    ASrep = jnp.concatenate([As3.reshape(HID, H1)] * 2, axis=1)
    ADrep = jnp.concatenate([Ad3.reshape(HID, H1)] * 2, axis=1)

    pidx = (jnp.arange(HID) % H1) * C1 + jnp.arange(HID) // H1
    gat1_Wp = gat1_W[:, pidx]
    ASrep_p = ASrep[pidx, :]
    ADrep_p = ADrep[pidx, :]
    g1b_p = gat1_b[pidx]
    gat2_Wp = gat2_W[pidx, :]
    As2rep = jnp.tile(gat2_asrc.reshape(NCLS, 1), (1, 16))
    Ad2rep = jnp.tile(gat2_adst.reshape(NCLS, 1), (1, 16))

    z144 = jnp.zeros((NP, W144), f32)
    z32 = jnp.zeros((NP, 32), f32)

    src_p2d = src_p.reshape(EP // G, G)
    dst_p2d = dst_p.reshape(EP // G, G)
    src2_2d = src2.reshape(EP // G, G)
    dst2_2d = dst2.reshape(EP // G, G)

    xs = _sc_gather(xp, src_p2d)

    nb = -(-E // BE)
    msg = pl.pallas_call(
        _tc_msg_body,
        grid=(nb,),
        in_specs=[
            pl.BlockSpec((BE, F_IN), lambda i: (i, 0)),
            pl.BlockSpec((BE, F_IN), lambda i: (i, 0)),
            pl.BlockSpec((F_IN, HID), lambda i: (0, 0)),
            pl.BlockSpec((1, HID), lambda i: (0, 0)),
            pl.BlockSpec((F_IN * HID, HID), lambda i: (0, 0)),
            pl.BlockSpec((F_IN, HID), lambda i: (0, 0)),
        ],
        out_specs=pl.BlockSpec((BE, W144), lambda i: (i, 0)),
        out_shape=jax.ShapeDtypeStruct((EP, W144), f32),
    )(edge_attr, xs, W1, b1r, W2r, B2)

    acc1 = _sc_scatter(msg, dst_p2d, z144).reshape(2, NP, W144)

    nb3 = NP // 2512
    thw1, tb1 = pl.pallas_call(
        _tc_h_gat1_body,
        grid=(nb3,),
        in_specs=[
            pl.BlockSpec((2, 2512, W144), lambda i: (0, i, 0)),
            pl.BlockSpec((2512, F_IN), lambda i: (i, 0)),
            pl.BlockSpec((F_IN, HID), lambda i: (0, 0)),
            pl.BlockSpec((1, HID), lambda i: (0, 0)),
            pl.BlockSpec((HID, HID), lambda i: (0, 0)),
            pl.BlockSpec((HID, 16), lambda i: (0, 0)),
            pl.BlockSpec((HID, 16), lambda i: (0, 0)),
        ],
        out_specs=[
            pl.BlockSpec((2512, W144), lambda i: (i, 0)),
            pl.BlockSpec((2512, 16), lambda i: (i, 0)),
        ],
        out_shape=[
            jax.ShapeDtypeStruct((NP, W144), f32),
            jax.ShapeDtypeStruct((NP, 16), f32),
        ],
    )(acc1, xp, Wroot, brootr, gat1_Wp, ASrep_p, ADrep_p)

    acc2 = _sc_gat(_sc_gat1_body, thw1, tb1, src2_2d, dst2_2d, z144,
                   W144).reshape(2, NP, W144)

    thw2, tb2 = pl.pallas_call(
        _tc_gat2_prep_body,
        grid=(nb3,),
        in_specs=[
            pl.BlockSpec((2, 2512, W144), lambda i: (0, i, 0)),
            pl.BlockSpec((1, HID), lambda i: (0, 0)),
            pl.BlockSpec((HID, NCLS), lambda i: (0, 0)),
            pl.BlockSpec((NCLS, 16), lambda i: (0, 0)),
            pl.BlockSpec((NCLS, 16), lambda i: (0, 0)),
        ],
        out_specs=[
            pl.BlockSpec((2512, 32), lambda i: (i, 0)),
            pl.BlockSpec((2512, 16), lambda i: (i, 0)),
        ],
        out_shape=[
            jax.ShapeDtypeStruct((NP, 32), f32),
            jax.ShapeDtypeStruct((NP, 16), f32),
        ],
    )(acc2, g1b_p.reshape(1, H1 * C1), gat2_Wp, As2rep, Ad2rep)

    acc3 = _sc_gat(_sc_gat2_body, thw2, tb2, src2_2d, dst2_2d, z32,
                   32).reshape(2, NP, 32)

    out = pl.pallas_call(
        _tc_final_body,
        grid=(nb3,),
        in_specs=[
            pl.BlockSpec((2, 2512, 32), lambda i: (0, i, 0)),
            pl.BlockSpec((1, NCLS), lambda i: (0, 0)),
        ],
        out_specs=pl.BlockSpec((2512, NCLS), lambda i: (i, 0)),
        out_shape=jax.ShapeDtypeStruct((NP, NCLS), f32),
    )(acc3, g2br)

    return out[:N]
